# Initial kernel scaffold; baseline (speedup 1.0000x reference)
import jax
import jax.numpy as jnp
from jax.experimental import pallas as pl


def kernel(h_atm, h_bnd, h_ang, params_A, params_G, edge_index_G, edge_index_A):
    raise NotImplementedError("write your pallas kernel here")


# speedup vs baseline: 1.8017x; 1.8017x over previous
import functools

import jax
import jax.numpy as jnp
from jax import lax
from jax.experimental import pallas as pl
from jax.experimental.pallas import tpu as pltpu
from jax.experimental.pallas import tpu_sc as plsc

D = 128
F32 = jnp.float32
I32 = jnp.int32


def _tc_pq(x, ws, wd):
    n = x.shape[0]
    bn = 512
    grid = (pl.cdiv(n, bn),)

    def body(x_ref, ws_ref, wd_ref, p_ref, q_ref):
        xb = x_ref[...]
        p_ref[...] = jnp.dot(xb, ws_ref[...], preferred_element_type=F32)
        q_ref[...] = jnp.dot(xb, wd_ref[...], preferred_element_type=F32)

    return pl.pallas_call(
        body,
        grid=grid,
        in_specs=[
            pl.BlockSpec((bn, D), lambda i: (i, 0)),
            pl.BlockSpec((D, D), lambda i: (0, 0)),
            pl.BlockSpec((D, D), lambda i: (0, 0)),
        ],
        out_specs=[pl.BlockSpec((bn, D), lambda i: (i, 0))] * 2,
        out_shape=[jax.ShapeDtypeStruct((n, D), F32)] * 2,
    )(x, ws, wd)


def _tc_edge(gp, gq, e, we, w2, b1, b2, g, bt):
    n = e.shape[0]
    bn = 1024
    grid = (pl.cdiv(n, bn),)

    def body(gp_ref, gq_ref, e_ref, we_ref, w2_ref, b1_ref, b2_ref, g_ref,
             bt_ref, out_ref):
        eb = e_ref[...]
        h = gp_ref[...] + gq_ref[...] + jnp.dot(
            eb, we_ref[...], preferred_element_type=F32) + b1_ref[...]
        h = jnp.maximum(h, 0.0)
        h = jnp.dot(h, w2_ref[...], preferred_element_type=F32) + b2_ref[...]
        mu = jnp.mean(h, axis=-1, keepdims=True)
        xc = h - mu
        var = jnp.mean(xc * xc, axis=-1, keepdims=True)
        h = xc * lax.rsqrt(var + 1e-5) * g_ref[...] + bt_ref[...]
        out_ref[...] = eb + h

    row = lambda i: (i, 0)
    fixed = lambda i: (0, 0)
    return pl.pallas_call(
        body,
        grid=grid,
        in_specs=[
            pl.BlockSpec((bn, D), row),
            pl.BlockSpec((bn, D), row),
            pl.BlockSpec((bn, D), row),
            pl.BlockSpec((D, D), fixed),
            pl.BlockSpec((D, D), fixed),
            pl.BlockSpec((1, D), fixed),
            pl.BlockSpec((1, D), fixed),
            pl.BlockSpec((1, D), fixed),
            pl.BlockSpec((1, D), fixed),
        ],
        out_specs=pl.BlockSpec((bn, D), row),
        out_shape=jax.ShapeDtypeStruct((n, D), F32),
    )(gp, gq, e, we, w2, b1, b2, g, bt)


def _tc_node(x, agg, w1a, w1b, w2, b1, b2, g, bt):
    n = x.shape[0]
    bn = 512
    grid = (pl.cdiv(n, bn),)

    def body(x_ref, a_ref, w1a_ref, w1b_ref, w2_ref, b1_ref, b2_ref, g_ref,
             bt_ref, out_ref):
        xb = x_ref[...]
        h = (jnp.dot(xb, w1a_ref[...], preferred_element_type=F32)
             + jnp.dot(a_ref[...], w1b_ref[...], preferred_element_type=F32)
             + b1_ref[...])
        h = jnp.maximum(h, 0.0)
        h = jnp.dot(h, w2_ref[...], preferred_element_type=F32) + b2_ref[...]
        mu = jnp.mean(h, axis=-1, keepdims=True)
        xc = h - mu
        var = jnp.mean(xc * xc, axis=-1, keepdims=True)
        h = xc * lax.rsqrt(var + 1e-5) * g_ref[...] + bt_ref[...]
        out_ref[...] = xb + h

    row = lambda i: (i, 0)
    fixed = lambda i: (0, 0)
    return pl.pallas_call(
        body,
        grid=grid,
        in_specs=[
            pl.BlockSpec((bn, D), row),
            pl.BlockSpec((bn, D), row),
            pl.BlockSpec((D, D), fixed),
            pl.BlockSpec((D, D), fixed),
            pl.BlockSpec((D, D), fixed),
            pl.BlockSpec((1, D), fixed),
            pl.BlockSpec((1, D), fixed),
            pl.BlockSpec((1, D), fixed),
            pl.BlockSpec((1, D), fixed),
        ],
        out_specs=pl.BlockSpec((bn, D), row),
        out_shape=jax.ShapeDtypeStruct((n, D), F32),
    )(x, agg, w1a, w1b, w2, b1, b2, g, bt)


_NC = 2
_NS = 16
_NW = _NC * _NS


def _pick_chunk(rows_per_worker, max_rows):
    ch = 8
    for cand in range(8, max_rows + 1, 8):
        if rows_per_worker % cand == 0:
            ch = cand
    return ch


@functools.cache
def _sc_gather2(e_count):
    rpw = e_count // _NW
    assert e_count % _NW == 0
    ch = _pick_chunk(rpw, 384)
    nsteps = rpw // ch
    mesh = plsc.VectorSubcoreMesh(core_axis_name="c", subcore_axis_name="s")

    @functools.partial(
        pl.kernel,
        mesh=mesh,
        out_type=[jax.ShapeDtypeStruct((e_count, D), F32)] * 2,
        scratch_types=[
            pltpu.VMEM((ch,), I32),
            pltpu.VMEM((ch,), I32),
            pltpu.VMEM((ch, D), F32),
            pltpu.VMEM((ch, D), F32),
            pltpu.SemaphoreType.DMA,
            pltpu.SemaphoreType.DMA,
        ],
    )
    def k(p_hbm, q_hbm, src_hbm, dst_hbm, gp_hbm, gq_hbm,
          idxs, idxd, bufp, bufq, sem1, sem2):
        wid = lax.axis_index("s") * _NC + lax.axis_index("c")
        base0 = wid * rpw

        def step(t, carry):
            base = base0 + t * ch
            pltpu.sync_copy(src_hbm.at[pl.ds(base, ch)], idxs)
            pltpu.sync_copy(dst_hbm.at[pl.ds(base, ch)], idxd)
            cp1 = pltpu.async_copy(p_hbm.at[idxs], bufp, sem1)
            cp2 = pltpu.async_copy(q_hbm.at[idxd], bufq, sem2)
            cp1.wait()
            cp2.wait()
            pltpu.sync_copy(bufp, gp_hbm.at[pl.ds(base, ch)])
            pltpu.sync_copy(bufq, gq_hbm.at[pl.ds(base, ch)])
            return carry

        lax.fori_loop(0, nsteps, step, 0)

    return k


_GC = 128


@functools.cache
def _sc_scatter_add(e_count, n_rows):
    ept = e_count // _NS
    assert e_count % _NS == 0 and ept % 16 == 0
    seg = 2000
    assert ept % seg == 0
    cap = ept + _GC + 16
    per_tile_words = seg + 2 * cap + _GC + _GC * D + 128 * D
    free_words = 2097151 - _NS * per_tile_words - 8192
    cmax = (free_words // D - 16) // 128 * 128
    if n_rows <= 2 * cmax:
        half = min(cmax, (n_rows // 2 + 127) // 128 * 128)
        sizes = [half, (n_rows - half + 127) // 128 * 128]
    else:
        sizes = [cmax] * (n_rows // cmax)
        rem = n_rows - cmax * len(sizes)
        if rem:
            sizes.append((rem + 127) // 128 * 128)
    n_pad = sum(sizes)
    chunks = []
    lo = 0
    for s in sizes:
        chunks.append((lo, s))
        lo += s
    nhalf = (len(chunks) + 1) // 2
    owners = [0 if i < nhalf else 1 for i in range(len(chunks))]
    cb = max(sizes) + 16

    mesh = plsc.VectorSubcoreMesh(core_axis_name="c", subcore_axis_name="s")

    @functools.partial(
        pl.kernel,
        mesh=mesh,
        compiler_params=pltpu.CompilerParams(needs_layout_passes=False),
        out_type=jax.ShapeDtypeStruct((n_pad, D), F32),
        scratch_types=[
            pltpu.VMEM((seg,), I32),
            pltpu.VMEM((cap,), I32),
            pltpu.VMEM((cap,), I32),
            pltpu.VMEM((_GC,), I32),
            pltpu.VMEM((_GC, D), F32),
            pltpu.VMEM((128, D), F32),
            pltpu.VMEM_SHARED((cb, D), F32),
            pltpu.SemaphoreType.DMA,
        ],
    )
    def k(rows_hbm, dst_hbm, out_hbm,
          dstseg, ids, offs, ostage, rowbuf, zbuf, accum, sem):
        cid = lax.axis_index("c")
        sid = lax.axis_index("s")
        ebase = sid * ept

        def zb_step(i, c):
            zbuf[i // 8, pl.ds((i % 8) * 16, 16)] = jnp.zeros((16,), F32)
            return c

        lax.fori_loop(0, 128 * 8, zb_step, 0)

        for (lo_c, size), owner in zip(chunks, owners):

            @pl.when(cid == owner)
            def _(lo_c=lo_c, size=size):
                z = size // _NS
                zoff = sid * z
                o = 0
                while o < z:
                    ln = min(128, z - o)
                    pltpu.sync_copy(zbuf.at[pl.ds(0, ln)],
                                    accum.at[pl.ds(zoff + o, ln)])
                    o += ln
                plsc.subcore_barrier()

                trash = cap - 16
                lane = lax.broadcasted_iota(I32, (16,), 0)

                def seg_step(si, cnt):
                    pltpu.sync_copy(
                        dst_hbm.at[pl.ds(ebase + si * seg, seg)], dstseg)

                    def scan_step(v, cnt):
                        dv = dstseg[pl.ds(v * 16, 16)]
                        m = (dv >= lo_c) & (dv < lo_c + size)
                        mi = m.astype(I32)
                        incl = plsc.cumsum(mi)
                        pos = jnp.where(m, cnt + incl - mi, trash)
                        gid = lane + (ebase + si * seg + v * 16)
                        plsc.store_scatter(ids, [pos], gid)
                        plsc.store_scatter(offs, [pos], dv - lo_c)
                        return cnt + jnp.sum(mi)

                    return lax.fori_loop(0, seg // 16, scan_step, cnt)

                cnt = lax.fori_loop(0, ept // seg, seg_step, 0)

                dummy = jnp.full((16,), size, I32)
                for j in range(_GC // 16):
                    ppos = lane + (cnt + j * 16)
                    plsc.store_scatter(ids, [ppos], jnp.zeros((16,), I32))
                    plsc.store_scatter(offs, [ppos], dummy)

                def gs_step(t, c):
                    for j in range(_GC // 16):
                        ostage[pl.ds(j * 16, 16)] = offs[
                            pl.ds(t * _GC + j * 16, 16)]
                    cp = pltpu.async_copy(
                        rows_hbm.at[ids.at[pl.ds(t * _GC, _GC)]], rowbuf, sem)
                    cp.wait()
                    pltpu.sync_copy(rowbuf, accum.at[ostage], add=True)
                    return c

                trips = (cnt + _GC - 1) // _GC
                lax.fori_loop(0, trips, gs_step, 0)
                plsc.subcore_barrier()

                w = size // _NS
                pltpu.sync_copy(accum.at[pl.ds(sid * w, w)],
                                out_hbm.at[pl.ds(lo_c + sid * w, w)])
                plsc.subcore_barrier()

    if n_pad == n_rows:
        return k
    return lambda rows, dst: k(rows, dst)[:n_rows]


def _mgn(x, e, src, dst, p, i):
    w1 = p["eW1"][i]
    ws, wd, we = w1[:D], w1[D:2 * D], w1[2 * D:]
    pt, qt = _tc_pq(x, ws, wd)
    gp, gq = _sc_gather2(e.shape[0])(pt, qt, src, dst)
    e_new = _tc_edge(gp, gq, e, we, p["eW2"][i],
                     p["eb1"][i].reshape(1, D), p["eb2"][i].reshape(1, D),
                     p["eg"][i].reshape(1, D), p["ebt"][i].reshape(1, D))
    agg = _sc_scatter_add(e.shape[0], x.shape[0])(e_new, dst)
    nw1 = p["nW1"][i]
    x_new = _tc_node(x, agg, nw1[:D], nw1[D:], p["nW2"][i],
                     p["nb1"][i].reshape(1, D), p["nb2"][i].reshape(1, D),
                     p["ng"][i].reshape(1, D), p["nbt"][i].reshape(1, D))
    return x_new, e_new


def kernel(h_atm, h_bnd, h_ang, params_A, params_G, edge_index_G, edge_index_A):
    srcA, dstA = edge_index_A[0], edge_index_A[1]
    srcG, dstG = edge_index_G[0], edge_index_G[1]
    for i in range(2):
        h_bnd, h_ang = _mgn(h_bnd, h_ang, srcA, dstA, params_A, i)
        h_atm, h_bnd = _mgn(h_atm, h_bnd, srcG, dstG, params_G, i)
    return (h_atm, h_bnd, h_ang)


# speedup vs baseline: 1.8511x; 1.0274x over previous
import functools

import jax
import jax.numpy as jnp
from jax import lax
from jax.experimental import pallas as pl
from jax.experimental.pallas import tpu as pltpu
from jax.experimental.pallas import tpu_sc as plsc

D = 128
F32 = jnp.float32
I32 = jnp.int32


def _tc_pq(x, ws, wd):
    n = x.shape[0]
    bn = 512
    grid = (pl.cdiv(n, bn),)

    def body(x_ref, ws_ref, wd_ref, p_ref, q_ref):
        xb = x_ref[...]
        p_ref[...] = jnp.dot(xb, ws_ref[...], preferred_element_type=F32)
        q_ref[...] = jnp.dot(xb, wd_ref[...], preferred_element_type=F32)

    return pl.pallas_call(
        body,
        grid=grid,
        in_specs=[
            pl.BlockSpec((bn, D), lambda i: (i, 0)),
            pl.BlockSpec((D, D), lambda i: (0, 0)),
            pl.BlockSpec((D, D), lambda i: (0, 0)),
        ],
        out_specs=[pl.BlockSpec((bn, D), lambda i: (i, 0))] * 2,
        out_shape=[jax.ShapeDtypeStruct((n, D), F32)] * 2,
    )(x, ws, wd)


def _tc_edge(gp, gq, e, we, w2, b1, b2, g, bt):
    n = e.shape[0]
    bn = 1024
    grid = (pl.cdiv(n, bn),)

    def body(gp_ref, gq_ref, e_ref, we_ref, w2_ref, b1_ref, b2_ref, g_ref,
             bt_ref, out_ref):
        eb = e_ref[...]
        h = gp_ref[...] + gq_ref[...] + jnp.dot(
            eb, we_ref[...], preferred_element_type=F32) + b1_ref[...]
        h = jnp.maximum(h, 0.0)
        h = jnp.dot(h, w2_ref[...], preferred_element_type=F32) + b2_ref[...]
        mu = jnp.mean(h, axis=-1, keepdims=True)
        xc = h - mu
        var = jnp.mean(xc * xc, axis=-1, keepdims=True)
        h = xc * lax.rsqrt(var + 1e-5) * g_ref[...] + bt_ref[...]
        out_ref[...] = eb + h

    row = lambda i: (i, 0)
    fixed = lambda i: (0, 0)
    return pl.pallas_call(
        body,
        grid=grid,
        in_specs=[
            pl.BlockSpec((bn, D), row),
            pl.BlockSpec((bn, D), row),
            pl.BlockSpec((bn, D), row),
            pl.BlockSpec((D, D), fixed),
            pl.BlockSpec((D, D), fixed),
            pl.BlockSpec((1, D), fixed),
            pl.BlockSpec((1, D), fixed),
            pl.BlockSpec((1, D), fixed),
            pl.BlockSpec((1, D), fixed),
        ],
        out_specs=pl.BlockSpec((bn, D), row),
        out_shape=jax.ShapeDtypeStruct((n, D), F32),
    )(gp, gq, e, we, w2, b1, b2, g, bt)


def _tc_node(x, agg, w1a, w1b, w2, b1, b2, g, bt):
    n = x.shape[0]
    bn = 512
    grid = (pl.cdiv(n, bn),)

    def body(x_ref, a_ref, w1a_ref, w1b_ref, w2_ref, b1_ref, b2_ref, g_ref,
             bt_ref, out_ref):
        xb = x_ref[...]
        h = (jnp.dot(xb, w1a_ref[...], preferred_element_type=F32)
             + jnp.dot(a_ref[...], w1b_ref[...], preferred_element_type=F32)
             + b1_ref[...])
        h = jnp.maximum(h, 0.0)
        h = jnp.dot(h, w2_ref[...], preferred_element_type=F32) + b2_ref[...]
        mu = jnp.mean(h, axis=-1, keepdims=True)
        xc = h - mu
        var = jnp.mean(xc * xc, axis=-1, keepdims=True)
        h = xc * lax.rsqrt(var + 1e-5) * g_ref[...] + bt_ref[...]
        out_ref[...] = xb + h

    row = lambda i: (i, 0)
    fixed = lambda i: (0, 0)
    return pl.pallas_call(
        body,
        grid=grid,
        in_specs=[
            pl.BlockSpec((bn, D), row),
            pl.BlockSpec((bn, D), row),
            pl.BlockSpec((D, D), fixed),
            pl.BlockSpec((D, D), fixed),
            pl.BlockSpec((D, D), fixed),
            pl.BlockSpec((1, D), fixed),
            pl.BlockSpec((1, D), fixed),
            pl.BlockSpec((1, D), fixed),
            pl.BlockSpec((1, D), fixed),
        ],
        out_specs=pl.BlockSpec((bn, D), row),
        out_shape=jax.ShapeDtypeStruct((n, D), F32),
    )(x, agg, w1a, w1b, w2, b1, b2, g, bt)


_NC = 2
_NS = 16
_NW = _NC * _NS


def _pick_chunk(rows_per_worker, max_rows):
    ch = 8
    for cand in range(8, max_rows + 1, 8):
        if rows_per_worker % cand == 0:
            ch = cand
    return ch


_NB = 2


@functools.cache
def _sc_gather2(e_count):
    rpw = e_count // _NW
    assert e_count % _NW == 0
    ch = _pick_chunk(rpw, 384)
    nsteps = rpw // ch
    assert nsteps >= _NB + 1
    mesh = plsc.VectorSubcoreMesh(core_axis_name="c", subcore_axis_name="s")

    scratch = []
    for _ in range(_NB):
        scratch += [pltpu.VMEM((ch,), I32), pltpu.VMEM((ch,), I32),
                    pltpu.VMEM((ch, D), F32), pltpu.VMEM((ch, D), F32)]
    scratch += [pltpu.SemaphoreType.DMA] * (6 * _NB)

    @functools.partial(
        pl.kernel,
        mesh=mesh,
        out_type=[jax.ShapeDtypeStruct((e_count, D), F32)] * 2,
        scratch_types=scratch,
    )
    def k(p_hbm, q_hbm, src_hbm, dst_hbm, gp_hbm, gq_hbm, *scr):
        bufs = [scr[4 * b:4 * b + 4] for b in range(_NB)]
        sems = scr[4 * _NB:]
        sem = lambda kind, b: sems[kind * _NB + b]

        wid = lax.axis_index("s") * _NC + lax.axis_index("c")
        base0 = wid * rpw

        def idx_cps(t, b):
            base = base0 + t * ch
            ixs, ixd = bufs[b][0], bufs[b][1]
            return (pltpu.make_async_copy(
                        src_hbm.at[pl.ds(base, ch)], ixs, sem(0, b)),
                    pltpu.make_async_copy(
                        dst_hbm.at[pl.ds(base, ch)], ixd, sem(1, b)))

        def gat_cps(b):
            ixs, ixd, bp, bq = bufs[b]
            return (pltpu.make_async_copy(p_hbm.at[ixs], bp, sem(2, b)),
                    pltpu.make_async_copy(q_hbm.at[ixd], bq, sem(3, b)))

        def wr_cps(t, b):
            base = base0 + t * ch
            bp, bq = bufs[b][2], bufs[b][3]
            return (pltpu.make_async_copy(
                        bp, gp_hbm.at[pl.ds(base, ch)], sem(4, b)),
                    pltpu.make_async_copy(
                        bq, gq_hbm.at[pl.ds(base, ch)], sem(5, b)))

        for b in range(_NB):
            for c in idx_cps(b, b):
                c.start()

        def outer(g, carry):
            t0 = g * _NB
            for b in range(_NB):
                t = t0 + b
                bprev = (b - 1) % _NB

                @pl.when(t < nsteps)
                def _(t=t, b=b, bprev=bprev):
                    for c in idx_cps(t, b):
                        c.wait()

                    @pl.when(t >= _NB)
                    def _():
                        for c in wr_cps(t - _NB, b):
                            c.wait()

                    for c in gat_cps(b):
                        c.start()

                    @pl.when(t >= 1)
                    def _():
                        for c in gat_cps(bprev):
                            c.wait()
                        for c in wr_cps(t - 1, bprev):
                            c.start()

                        @pl.when(t - 1 + _NB < nsteps)
                        def _():
                            for c in idx_cps(t - 1 + _NB, bprev):
                                c.start()
            return carry

        lax.fori_loop(0, pl.cdiv(nsteps, _NB), outer, 0)

        tl = nsteps - 1
        for c in gat_cps(tl % _NB):
            c.wait()
        for c in wr_cps(tl, tl % _NB):
            c.start()
        for t in range(nsteps - _NB, nsteps):
            for c in wr_cps(t, t % _NB):
                c.wait()

    return k


_GC = 128


@functools.cache
def _sc_scatter_add(e_count, n_rows):
    ept = e_count // _NS
    assert e_count % _NS == 0 and ept % 16 == 0
    seg = 2000
    assert ept % seg == 0
    cap = ept + _GC + 16
    per_tile_words = seg + 2 * cap + _GC + _GC * D + 128 * D
    free_words = 2097151 - _NS * per_tile_words - 8192
    cmax = (free_words // D - 16) // 128 * 128
    if n_rows <= 2 * cmax:
        half = min(cmax, (n_rows // 2 + 127) // 128 * 128)
        sizes = [half, (n_rows - half + 127) // 128 * 128]
    else:
        sizes = [cmax] * (n_rows // cmax)
        rem = n_rows - cmax * len(sizes)
        if rem:
            sizes.append((rem + 127) // 128 * 128)
    n_pad = sum(sizes)
    chunks = []
    lo = 0
    for s in sizes:
        chunks.append((lo, s))
        lo += s
    nhalf = (len(chunks) + 1) // 2
    owners = [0 if i < nhalf else 1 for i in range(len(chunks))]
    cb = max(sizes) + 16

    mesh = plsc.VectorSubcoreMesh(core_axis_name="c", subcore_axis_name="s")

    @functools.partial(
        pl.kernel,
        mesh=mesh,
        compiler_params=pltpu.CompilerParams(needs_layout_passes=False),
        out_type=jax.ShapeDtypeStruct((n_pad, D), F32),
        scratch_types=[
            pltpu.VMEM((seg,), I32),
            pltpu.VMEM((cap,), I32),
            pltpu.VMEM((cap,), I32),
            pltpu.VMEM((_GC,), I32),
            pltpu.VMEM((_GC, D), F32),
            pltpu.VMEM((128, D), F32),
            pltpu.VMEM_SHARED((cb, D), F32),
            pltpu.SemaphoreType.DMA,
        ],
    )
    def k(rows_hbm, dst_hbm, out_hbm,
          dstseg, ids, offs, ostage, rowbuf, zbuf, accum, sem):
        cid = lax.axis_index("c")
        sid = lax.axis_index("s")
        ebase = sid * ept

        def zb_step(i, c):
            zbuf[i // 8, pl.ds((i % 8) * 16, 16)] = jnp.zeros((16,), F32)
            return c

        lax.fori_loop(0, 128 * 8, zb_step, 0)

        for (lo_c, size), owner in zip(chunks, owners):

            @pl.when(cid == owner)
            def _(lo_c=lo_c, size=size):
                z = size // _NS
                zoff = sid * z
                o = 0
                while o < z:
                    ln = min(128, z - o)
                    pltpu.sync_copy(zbuf.at[pl.ds(0, ln)],
                                    accum.at[pl.ds(zoff + o, ln)])
                    o += ln
                plsc.subcore_barrier()

                trash = cap - 16
                lane = lax.broadcasted_iota(I32, (16,), 0)

                def seg_step(si, cnt):
                    pltpu.sync_copy(
                        dst_hbm.at[pl.ds(ebase + si * seg, seg)], dstseg)

                    def scan_step(v, cnt):
                        dv = dstseg[pl.ds(v * 16, 16)]
                        m = (dv >= lo_c) & (dv < lo_c + size)
                        mi = m.astype(I32)
                        incl = plsc.cumsum(mi)
                        pos = jnp.where(m, cnt + incl - mi, trash)
                        gid = lane + (ebase + si * seg + v * 16)
                        plsc.store_scatter(ids, [pos], gid)
                        plsc.store_scatter(offs, [pos], dv - lo_c)
                        return cnt + jnp.sum(mi)

                    return lax.fori_loop(0, seg // 16, scan_step, cnt)

                cnt = lax.fori_loop(0, ept // seg, seg_step, 0)

                dummy = jnp.full((16,), size, I32)
                for j in range(_GC // 16):
                    ppos = lane + (cnt + j * 16)
                    plsc.store_scatter(ids, [ppos], jnp.zeros((16,), I32))
                    plsc.store_scatter(offs, [ppos], dummy)

                def gs_step(t, c):
                    for j in range(_GC // 16):
                        ostage[pl.ds(j * 16, 16)] = offs[
                            pl.ds(t * _GC + j * 16, 16)]
                    cp = pltpu.async_copy(
                        rows_hbm.at[ids.at[pl.ds(t * _GC, _GC)]], rowbuf, sem)
                    cp.wait()
                    pltpu.sync_copy(rowbuf, accum.at[ostage], add=True)
                    return c

                trips = (cnt + _GC - 1) // _GC
                lax.fori_loop(0, trips, gs_step, 0)
                plsc.subcore_barrier()

                w = size // _NS
                pltpu.sync_copy(accum.at[pl.ds(sid * w, w)],
                                out_hbm.at[pl.ds(lo_c + sid * w, w)])
                plsc.subcore_barrier()

    if n_pad == n_rows:
        return k
    return lambda rows, dst: k(rows, dst)[:n_rows]


def _mgn(x, e, src, dst, p, i):
    w1 = p["eW1"][i]
    ws, wd, we = w1[:D], w1[D:2 * D], w1[2 * D:]
    pt, qt = _tc_pq(x, ws, wd)
    gp, gq = _sc_gather2(e.shape[0])(pt, qt, src, dst)
    e_new = _tc_edge(gp, gq, e, we, p["eW2"][i],
                     p["eb1"][i].reshape(1, D), p["eb2"][i].reshape(1, D),
                     p["eg"][i].reshape(1, D), p["ebt"][i].reshape(1, D))
    agg = _sc_scatter_add(e.shape[0], x.shape[0])(e_new, dst)
    nw1 = p["nW1"][i]
    x_new = _tc_node(x, agg, nw1[:D], nw1[D:], p["nW2"][i],
                     p["nb1"][i].reshape(1, D), p["nb2"][i].reshape(1, D),
                     p["ng"][i].reshape(1, D), p["nbt"][i].reshape(1, D))
    return x_new, e_new


def kernel(h_atm, h_bnd, h_ang, params_A, params_G, edge_index_G, edge_index_A):
    srcA, dstA = edge_index_A[0], edge_index_A[1]
    srcG, dstG = edge_index_G[0], edge_index_G[1]
    for i in range(2):
        h_bnd, h_ang = _mgn(h_bnd, h_ang, srcA, dstA, params_A, i)
        h_atm, h_bnd = _mgn(h_atm, h_bnd, srcG, dstG, params_G, i)
    return (h_atm, h_bnd, h_ang)


# speedup vs baseline: 1.8655x; 1.0078x over previous
import functools

import jax
import jax.numpy as jnp
from jax import lax
from jax.experimental import pallas as pl
from jax.experimental.pallas import tpu as pltpu
from jax.experimental.pallas import tpu_sc as plsc

D = 128
F32 = jnp.float32
I32 = jnp.int32


def _tc_pq(x, ws, wd):
    n = x.shape[0]
    bn = 512
    grid = (pl.cdiv(n, bn),)

    def body(x_ref, ws_ref, wd_ref, p_ref, q_ref):
        xb = x_ref[...]
        p_ref[...] = jnp.dot(xb, ws_ref[...], preferred_element_type=F32)
        q_ref[...] = jnp.dot(xb, wd_ref[...], preferred_element_type=F32)

    return pl.pallas_call(
        body,
        grid=grid,
        in_specs=[
            pl.BlockSpec((bn, D), lambda i: (i, 0)),
            pl.BlockSpec((D, D), lambda i: (0, 0)),
            pl.BlockSpec((D, D), lambda i: (0, 0)),
        ],
        out_specs=[pl.BlockSpec((bn, D), lambda i: (i, 0))] * 2,
        out_shape=[jax.ShapeDtypeStruct((n, D), F32)] * 2,
    )(x, ws, wd)


def _tc_edge(gp, gq, e, we, w2, b1, b2, g, bt):
    n = e.shape[0]
    bn = 1024
    grid = (pl.cdiv(n, bn),)

    def body(gp_ref, gq_ref, e_ref, we_ref, w2_ref, b1_ref, b2_ref, g_ref,
             bt_ref, out_ref):
        eb = e_ref[...]
        h = gp_ref[...] + gq_ref[...] + jnp.dot(
            eb, we_ref[...], preferred_element_type=F32) + b1_ref[...]
        h = jnp.maximum(h, 0.0)
        h = jnp.dot(h, w2_ref[...], preferred_element_type=F32) + b2_ref[...]
        mu = jnp.mean(h, axis=-1, keepdims=True)
        xc = h - mu
        var = jnp.mean(xc * xc, axis=-1, keepdims=True)
        h = xc * lax.rsqrt(var + 1e-5) * g_ref[...] + bt_ref[...]
        out_ref[...] = eb + h

    row = lambda i: (i, 0)
    fixed = lambda i: (0, 0)
    return pl.pallas_call(
        body,
        grid=grid,
        in_specs=[
            pl.BlockSpec((bn, D), row),
            pl.BlockSpec((bn, D), row),
            pl.BlockSpec((bn, D), row),
            pl.BlockSpec((D, D), fixed),
            pl.BlockSpec((D, D), fixed),
            pl.BlockSpec((1, D), fixed),
            pl.BlockSpec((1, D), fixed),
            pl.BlockSpec((1, D), fixed),
            pl.BlockSpec((1, D), fixed),
        ],
        out_specs=pl.BlockSpec((bn, D), row),
        out_shape=jax.ShapeDtypeStruct((n, D), F32),
    )(gp, gq, e, we, w2, b1, b2, g, bt)


def _tc_node(x, agg, w1a, w1b, w2, b1, b2, g, bt):
    n = x.shape[0]
    bn = 512
    grid = (pl.cdiv(n, bn),)

    def body(x_ref, a_ref, w1a_ref, w1b_ref, w2_ref, b1_ref, b2_ref, g_ref,
             bt_ref, out_ref):
        xb = x_ref[...]
        h = (jnp.dot(xb, w1a_ref[...], preferred_element_type=F32)
             + jnp.dot(a_ref[...], w1b_ref[...], preferred_element_type=F32)
             + b1_ref[...])
        h = jnp.maximum(h, 0.0)
        h = jnp.dot(h, w2_ref[...], preferred_element_type=F32) + b2_ref[...]
        mu = jnp.mean(h, axis=-1, keepdims=True)
        xc = h - mu
        var = jnp.mean(xc * xc, axis=-1, keepdims=True)
        h = xc * lax.rsqrt(var + 1e-5) * g_ref[...] + bt_ref[...]
        out_ref[...] = xb + h

    row = lambda i: (i, 0)
    fixed = lambda i: (0, 0)
    return pl.pallas_call(
        body,
        grid=grid,
        in_specs=[
            pl.BlockSpec((bn, D), row),
            pl.BlockSpec((bn, D), row),
            pl.BlockSpec((D, D), fixed),
            pl.BlockSpec((D, D), fixed),
            pl.BlockSpec((D, D), fixed),
            pl.BlockSpec((1, D), fixed),
            pl.BlockSpec((1, D), fixed),
            pl.BlockSpec((1, D), fixed),
            pl.BlockSpec((1, D), fixed),
        ],
        out_specs=pl.BlockSpec((bn, D), row),
        out_shape=jax.ShapeDtypeStruct((n, D), F32),
    )(x, agg, w1a, w1b, w2, b1, b2, g, bt)


_NC = 2
_NS = 16
_NW = _NC * _NS


def _pick_chunk(rows_per_worker, max_rows):
    ch = 8
    for cand in range(8, max_rows + 1, 8):
        if rows_per_worker % cand == 0:
            ch = cand
    return ch


_NB = 2


@functools.cache
def _sc_gather2(e_count):
    rpw = e_count // _NW
    assert e_count % _NW == 0
    ch = _pick_chunk(rpw, 384)
    nsteps = rpw // ch
    assert nsteps >= _NB + 1
    mesh = plsc.VectorSubcoreMesh(core_axis_name="c", subcore_axis_name="s")

    scratch = []
    for _ in range(_NB):
        scratch += [pltpu.VMEM((ch,), I32), pltpu.VMEM((ch,), I32),
                    pltpu.VMEM((ch, D), F32), pltpu.VMEM((ch, D), F32)]
    scratch += [pltpu.SemaphoreType.DMA] * (6 * _NB)

    @functools.partial(
        pl.kernel,
        mesh=mesh,
        out_type=[jax.ShapeDtypeStruct((e_count, D), F32)] * 2,
        scratch_types=scratch,
    )
    def k(p_hbm, q_hbm, src_hbm, dst_hbm, gp_hbm, gq_hbm, *scr):
        bufs = [scr[4 * b:4 * b + 4] for b in range(_NB)]
        sems = scr[4 * _NB:]
        sem = lambda kind, b: sems[kind * _NB + b]

        wid = lax.axis_index("s") * _NC + lax.axis_index("c")
        base0 = wid * rpw

        def idx_cps(t, b):
            base = base0 + t * ch
            ixs, ixd = bufs[b][0], bufs[b][1]
            return (pltpu.make_async_copy(
                        src_hbm.at[pl.ds(base, ch)], ixs, sem(0, b)),
                    pltpu.make_async_copy(
                        dst_hbm.at[pl.ds(base, ch)], ixd, sem(1, b)))

        def gat_cps(b):
            ixs, ixd, bp, bq = bufs[b]
            return (pltpu.make_async_copy(p_hbm.at[ixs], bp, sem(2, b)),
                    pltpu.make_async_copy(q_hbm.at[ixd], bq, sem(3, b)))

        def wr_cps(t, b):
            base = base0 + t * ch
            bp, bq = bufs[b][2], bufs[b][3]
            return (pltpu.make_async_copy(
                        bp, gp_hbm.at[pl.ds(base, ch)], sem(4, b)),
                    pltpu.make_async_copy(
                        bq, gq_hbm.at[pl.ds(base, ch)], sem(5, b)))

        for b in range(_NB):
            for c in idx_cps(b, b):
                c.start()

        def outer(g, carry):
            t0 = g * _NB
            for b in range(_NB):
                t = t0 + b
                bprev = (b - 1) % _NB

                @pl.when(t < nsteps)
                def _(t=t, b=b, bprev=bprev):
                    for c in idx_cps(t, b):
                        c.wait()

                    @pl.when(t >= _NB)
                    def _():
                        for c in wr_cps(t - _NB, b):
                            c.wait()

                    for c in gat_cps(b):
                        c.start()

                    @pl.when(t >= 1)
                    def _():
                        for c in gat_cps(bprev):
                            c.wait()
                        for c in wr_cps(t - 1, bprev):
                            c.start()

                        @pl.when(t - 1 + _NB < nsteps)
                        def _():
                            for c in idx_cps(t - 1 + _NB, bprev):
                                c.start()
            return carry

        lax.fori_loop(0, pl.cdiv(nsteps, _NB), outer, 0)

        tl = nsteps - 1
        for c in gat_cps(tl % _NB):
            c.wait()
        for c in wr_cps(tl, tl % _NB):
            c.start()
        for t in range(nsteps - _NB, nsteps):
            for c in wr_cps(t, t % _NB):
                c.wait()

    return k


_GC = 128


@functools.cache
def _sc_scatter_add(e_count, n_rows):
    ept = e_count // _NS
    assert e_count % _NS == 0 and ept % 16 == 0
    seg = 2000
    assert ept % seg == 0
    cap = ept + _GC + 16
    per_tile_words = seg + 2 * cap + 2 * _GC + 2 * _GC * D
    free_words = 2097151 - _NS * per_tile_words - 8192
    cmax = (free_words // D - 16) // 128 * 128
    if n_rows <= 2 * cmax:
        half = min(cmax, (n_rows // 2 + 127) // 128 * 128)
        sizes = [half, (n_rows - half + 127) // 128 * 128]
    else:
        sizes = [cmax] * (n_rows // cmax)
        rem = n_rows - cmax * len(sizes)
        if rem:
            sizes.append((rem + 127) // 128 * 128)
    n_pad = sum(sizes)
    chunks = []
    lo = 0
    for s in sizes:
        chunks.append((lo, s))
        lo += s
    nhalf = (len(chunks) + 1) // 2
    owners = [0 if i < nhalf else 1 for i in range(len(chunks))]
    cb = max(sizes) + 16

    mesh = plsc.VectorSubcoreMesh(core_axis_name="c", subcore_axis_name="s")

    @functools.partial(
        pl.kernel,
        mesh=mesh,
        compiler_params=pltpu.CompilerParams(needs_layout_passes=False),
        out_type=jax.ShapeDtypeStruct((n_pad, D), F32),
        scratch_types=[
            pltpu.VMEM((seg,), I32),
            pltpu.VMEM((cap,), I32),
            pltpu.VMEM((cap,), I32),
            pltpu.VMEM((_GC,), I32),
            pltpu.VMEM((_GC,), I32),
            pltpu.VMEM((_GC, D), F32),
            pltpu.VMEM((_GC, D), F32),
            pltpu.VMEM_SHARED((cb, D), F32),
            pltpu.SemaphoreType.DMA,
            pltpu.SemaphoreType.DMA,
        ],
    )
    def k(rows_hbm, dst_hbm, out_hbm,
          dstseg, ids, offs, ost0, ost1, rb0, rb1, accum, sem0, sem1):
        cid = lax.axis_index("c")
        sid = lax.axis_index("s")
        ebase = sid * ept
        ostages = (ost0, ost1)
        rowbufs = (rb0, rb1)
        sems = (sem0, sem1)

        for (lo_c, size), owner in zip(chunks, owners):

            @pl.when(cid == owner)
            def _(lo_c=lo_c, size=size):
                def zb_step(i, c):
                    rb0[i // 8, pl.ds((i % 8) * 16, 16)] = jnp.zeros((16,), F32)
                    return c

                lax.fori_loop(0, _GC * 8, zb_step, 0)
                z = size // _NS
                zoff = sid * z
                o = 0
                while o < z:
                    ln = min(_GC, z - o)
                    pltpu.sync_copy(rb0.at[pl.ds(0, ln)],
                                    accum.at[pl.ds(zoff + o, ln)])
                    o += ln
                plsc.subcore_barrier()

                trash = cap - 16
                lane = lax.broadcasted_iota(I32, (16,), 0)

                def seg_step(si, cnt):
                    pltpu.sync_copy(
                        dst_hbm.at[pl.ds(ebase + si * seg, seg)], dstseg)

                    def scan_step(v, cnt):
                        dv = dstseg[pl.ds(v * 16, 16)]
                        m = (dv >= lo_c) & (dv < lo_c + size)
                        mi = m.astype(I32)
                        incl = plsc.cumsum(mi)
                        pos = jnp.where(m, cnt + incl - mi, trash)
                        gid = lane + (ebase + si * seg + v * 16)
                        plsc.store_scatter(ids, [pos], gid)
                        plsc.store_scatter(offs, [pos], dv - lo_c)
                        return cnt + jnp.sum(mi)

                    return lax.fori_loop(0, seg // 16, scan_step, cnt)

                cnt = lax.fori_loop(0, ept // seg, seg_step, 0)

                dummy = jnp.full((16,), size, I32)
                for j in range(_GC // 16):
                    ppos = lane + (cnt + j * 16)
                    plsc.store_scatter(ids, [ppos], jnp.zeros((16,), I32))
                    plsc.store_scatter(offs, [ppos], dummy)

                trips = (cnt + _GC - 1) // _GC

                def gat(t, b):
                    return pltpu.make_async_copy(
                        rows_hbm.at[ids.at[pl.ds(t * _GC, _GC)]],
                        rowbufs[b], sems[b])

                @pl.when(trips > 0)
                def _():
                    gat(0, 0).start()

                def gs_outer(g, c):
                    for b in range(2):
                        t = g * 2 + b

                        @pl.when(t < trips)
                        def _(t=t, b=b):
                            gat(t, b).wait()

                            @pl.when(t + 1 < trips)
                            def _():
                                gat(t + 1, 1 - b).start()

                            for j in range(_GC // 16):
                                ostages[b][pl.ds(j * 16, 16)] = offs[
                                    pl.ds(t * _GC + j * 16, 16)]
                            pltpu.sync_copy(rowbufs[b], accum.at[ostages[b]],
                                            add=True)
                    return c

                lax.fori_loop(0, (trips + 1) // 2, gs_outer, 0)
                plsc.subcore_barrier()

                w = size // _NS
                pltpu.sync_copy(accum.at[pl.ds(sid * w, w)],
                                out_hbm.at[pl.ds(lo_c + sid * w, w)])
                plsc.subcore_barrier()

    if n_pad == n_rows:
        return k
    return lambda rows, dst: k(rows, dst)[:n_rows]


def _mgn(x, e, src, dst, p, i):
    w1 = p["eW1"][i]
    ws, wd, we = w1[:D], w1[D:2 * D], w1[2 * D:]
    pt, qt = _tc_pq(x, ws, wd)
    gp, gq = _sc_gather2(e.shape[0])(pt, qt, src, dst)
    e_new = _tc_edge(gp, gq, e, we, p["eW2"][i],
                     p["eb1"][i].reshape(1, D), p["eb2"][i].reshape(1, D),
                     p["eg"][i].reshape(1, D), p["ebt"][i].reshape(1, D))
    agg = _sc_scatter_add(e.shape[0], x.shape[0])(e_new, dst)
    nw1 = p["nW1"][i]
    x_new = _tc_node(x, agg, nw1[:D], nw1[D:], p["nW2"][i],
                     p["nb1"][i].reshape(1, D), p["nb2"][i].reshape(1, D),
                     p["ng"][i].reshape(1, D), p["nbt"][i].reshape(1, D))
    return x_new, e_new


def kernel(h_atm, h_bnd, h_ang, params_A, params_G, edge_index_G, edge_index_A):
    srcA, dstA = edge_index_A[0], edge_index_A[1]
    srcG, dstG = edge_index_G[0], edge_index_G[1]
    for i in range(2):
        h_bnd, h_ang = _mgn(h_bnd, h_ang, srcA, dstA, params_A, i)
        h_atm, h_bnd = _mgn(h_atm, h_bnd, srcG, dstG, params_G, i)
    return (h_atm, h_bnd, h_ang)


# speedup vs baseline: 1.9938x; 1.0688x over previous
import functools

import jax
import jax.numpy as jnp
from jax import lax
from jax.experimental import pallas as pl
from jax.experimental.pallas import tpu as pltpu
from jax.experimental.pallas import tpu_sc as plsc

D = 128
F32 = jnp.float32
I32 = jnp.int32


def _tc_pq(x, ws, wd):
    n = x.shape[0]
    bn = 512
    grid = (pl.cdiv(n, bn),)

    def body(x_ref, ws_ref, wd_ref, p_ref, q_ref):
        xb = x_ref[...]
        p_ref[...] = jnp.dot(xb, ws_ref[...], preferred_element_type=F32)
        q_ref[...] = jnp.dot(xb, wd_ref[...], preferred_element_type=F32)

    return pl.pallas_call(
        body,
        grid=grid,
        in_specs=[
            pl.BlockSpec((bn, D), lambda i: (i, 0)),
            pl.BlockSpec((D, D), lambda i: (0, 0)),
            pl.BlockSpec((D, D), lambda i: (0, 0)),
        ],
        out_specs=[pl.BlockSpec((bn, D), lambda i: (i, 0))] * 2,
        out_shape=[jax.ShapeDtypeStruct((n, D), F32)] * 2,
    )(x, ws, wd)


def _tc_edge(gp, gq, e, we, w2, b1, b2, g, bt):
    n = e.shape[0]
    bn = 1024
    grid = (pl.cdiv(n, bn),)

    def body(gp_ref, gq_ref, e_ref, we_ref, w2_ref, b1_ref, b2_ref, g_ref,
             bt_ref, out_ref):
        eb = e_ref[...]
        h = gp_ref[...] + gq_ref[...] + jnp.dot(
            eb, we_ref[...], preferred_element_type=F32) + b1_ref[...]
        h = jnp.maximum(h, 0.0)
        h = jnp.dot(h, w2_ref[...], preferred_element_type=F32) + b2_ref[...]
        mu = jnp.mean(h, axis=-1, keepdims=True)
        xc = h - mu
        var = jnp.mean(xc * xc, axis=-1, keepdims=True)
        h = xc * lax.rsqrt(var + 1e-5) * g_ref[...] + bt_ref[...]
        out_ref[...] = eb + h

    row = lambda i: (i, 0)
    fixed = lambda i: (0, 0)
    return pl.pallas_call(
        body,
        grid=grid,
        in_specs=[
            pl.BlockSpec((bn, D), row),
            pl.BlockSpec((bn, D), row),
            pl.BlockSpec((bn, D), row),
            pl.BlockSpec((D, D), fixed),
            pl.BlockSpec((D, D), fixed),
            pl.BlockSpec((1, D), fixed),
            pl.BlockSpec((1, D), fixed),
            pl.BlockSpec((1, D), fixed),
            pl.BlockSpec((1, D), fixed),
        ],
        out_specs=pl.BlockSpec((bn, D), row),
        out_shape=jax.ShapeDtypeStruct((n, D), F32),
    )(gp, gq, e, we, w2, b1, b2, g, bt)


def _tc_node(x, agg, w1a, w1b, w2, b1, b2, g, bt):
    n = x.shape[0]
    bn = 512
    grid = (pl.cdiv(n, bn),)

    def body(x_ref, a_ref, w1a_ref, w1b_ref, w2_ref, b1_ref, b2_ref, g_ref,
             bt_ref, out_ref):
        xb = x_ref[...]
        h = (jnp.dot(xb, w1a_ref[...], preferred_element_type=F32)
             + jnp.dot(a_ref[...], w1b_ref[...], preferred_element_type=F32)
             + b1_ref[...])
        h = jnp.maximum(h, 0.0)
        h = jnp.dot(h, w2_ref[...], preferred_element_type=F32) + b2_ref[...]
        mu = jnp.mean(h, axis=-1, keepdims=True)
        xc = h - mu
        var = jnp.mean(xc * xc, axis=-1, keepdims=True)
        h = xc * lax.rsqrt(var + 1e-5) * g_ref[...] + bt_ref[...]
        out_ref[...] = xb + h

    row = lambda i: (i, 0)
    fixed = lambda i: (0, 0)
    return pl.pallas_call(
        body,
        grid=grid,
        in_specs=[
            pl.BlockSpec((bn, D), row),
            pl.BlockSpec((bn, D), row),
            pl.BlockSpec((D, D), fixed),
            pl.BlockSpec((D, D), fixed),
            pl.BlockSpec((D, D), fixed),
            pl.BlockSpec((1, D), fixed),
            pl.BlockSpec((1, D), fixed),
            pl.BlockSpec((1, D), fixed),
            pl.BlockSpec((1, D), fixed),
        ],
        out_specs=pl.BlockSpec((bn, D), row),
        out_shape=jax.ShapeDtypeStruct((n, D), F32),
    )(x, agg, w1a, w1b, w2, b1, b2, g, bt)


_NC = 2
_NS = 16
_NW = _NC * _NS


def _pick_chunk(rows_per_worker, max_rows):
    ch = 8
    for cand in range(8, max_rows + 1, 8):
        if rows_per_worker % cand == 0:
            ch = cand
    return ch


_NB = 2


@functools.cache
def _sc_gather2(e_count):
    rpw = e_count // _NW
    assert e_count % _NW == 0
    ch = _pick_chunk(rpw, 384)
    nsteps = rpw // ch
    assert nsteps >= _NB + 1
    mesh = plsc.VectorSubcoreMesh(core_axis_name="c", subcore_axis_name="s")

    scratch = []
    for _ in range(_NB):
        scratch += [pltpu.VMEM((ch,), I32), pltpu.VMEM((ch,), I32),
                    pltpu.VMEM((ch, D), F32), pltpu.VMEM((ch, D), F32)]
    scratch += [pltpu.SemaphoreType.DMA] * (6 * _NB)

    @functools.partial(
        pl.kernel,
        mesh=mesh,
        out_type=[jax.ShapeDtypeStruct((e_count, D), F32)] * 2,
        scratch_types=scratch,
    )
    def k(p_hbm, q_hbm, src_hbm, dst_hbm, gp_hbm, gq_hbm, *scr):
        bufs = [scr[4 * b:4 * b + 4] for b in range(_NB)]
        sems = scr[4 * _NB:]
        sem = lambda kind, b: sems[kind * _NB + b]

        wid = lax.axis_index("s") * _NC + lax.axis_index("c")
        base0 = wid * rpw

        def idx_cps(t, b):
            base = base0 + t * ch
            ixs, ixd = bufs[b][0], bufs[b][1]
            return (pltpu.make_async_copy(
                        src_hbm.at[pl.ds(base, ch)], ixs, sem(0, b)),
                    pltpu.make_async_copy(
                        dst_hbm.at[pl.ds(base, ch)], ixd, sem(1, b)))

        def gat_cps(b):
            ixs, ixd, bp, bq = bufs[b]
            return (pltpu.make_async_copy(p_hbm.at[ixs], bp, sem(2, b)),
                    pltpu.make_async_copy(q_hbm.at[ixd], bq, sem(3, b)))

        def wr_cps(t, b):
            base = base0 + t * ch
            bp, bq = bufs[b][2], bufs[b][3]
            return (pltpu.make_async_copy(
                        bp, gp_hbm.at[pl.ds(base, ch)], sem(4, b)),
                    pltpu.make_async_copy(
                        bq, gq_hbm.at[pl.ds(base, ch)], sem(5, b)))

        for b in range(_NB):
            for c in idx_cps(b, b):
                c.start()

        def outer(g, carry):
            t0 = g * _NB
            for b in range(_NB):
                t = t0 + b
                bprev = (b - 1) % _NB

                @pl.when(t < nsteps)
                def _(t=t, b=b, bprev=bprev):
                    for c in idx_cps(t, b):
                        c.wait()

                    @pl.when(t >= _NB)
                    def _():
                        for c in wr_cps(t - _NB, b):
                            c.wait()

                    for c in gat_cps(b):
                        c.start()

                    @pl.when(t >= 1)
                    def _():
                        for c in gat_cps(bprev):
                            c.wait()
                        for c in wr_cps(t - 1, bprev):
                            c.start()

                        @pl.when(t - 1 + _NB < nsteps)
                        def _():
                            for c in idx_cps(t - 1 + _NB, bprev):
                                c.start()
            return carry

        lax.fori_loop(0, pl.cdiv(nsteps, _NB), outer, 0)

        tl = nsteps - 1
        for c in gat_cps(tl % _NB):
            c.wait()
        for c in wr_cps(tl, tl % _NB):
            c.start()
        for t in range(nsteps - _NB, nsteps):
            for c in wr_cps(t, t % _NB):
                c.wait()

    return k


_GC = 128


@functools.cache
def _sc_scatter_add(e_count, n_rows):
    ept = e_count // _NS
    assert e_count % _NS == 0 and ept % 16 == 0
    seg = 2000
    assert ept % seg == 0
    cap = 6144
    assert cap - seg - _GC > seg
    per_tile_words = seg + 2 * cap + 2 * _GC + 2 * _GC * D
    free_words = 2097151 - _NS * per_tile_words - 8192
    cmax = (free_words // D - 16) // 128 * 128
    if n_rows <= 2 * cmax:
        half = min(cmax, (n_rows // 2 + 127) // 128 * 128)
        sizes = [half, (n_rows - half + 127) // 128 * 128]
    else:
        sizes = [cmax] * (n_rows // cmax)
        rem = n_rows - cmax * len(sizes)
        if rem:
            sizes.append((rem + 127) // 128 * 128)
    n_pad = sum(sizes)
    chunks = []
    lo = 0
    for s in sizes:
        chunks.append((lo, s))
        lo += s
    nhalf = (len(chunks) + 1) // 2
    owners = [0 if i < nhalf else 1 for i in range(len(chunks))]
    cb = max(sizes) + 16

    mesh = plsc.VectorSubcoreMesh(core_axis_name="c", subcore_axis_name="s")

    @functools.partial(
        pl.kernel,
        mesh=mesh,
        compiler_params=pltpu.CompilerParams(needs_layout_passes=False),
        out_type=jax.ShapeDtypeStruct((n_pad, D), F32),
        scratch_types=[
            pltpu.VMEM((seg,), I32),
            pltpu.VMEM((cap,), I32),
            pltpu.VMEM((cap,), I32),
            pltpu.VMEM((_GC,), I32),
            pltpu.VMEM((_GC,), I32),
            pltpu.VMEM((_GC, D), F32),
            pltpu.VMEM((_GC, D), F32),
            pltpu.VMEM_SHARED((cb, D), F32),
            pltpu.SemaphoreType.DMA,
            pltpu.SemaphoreType.DMA,
        ],
    )
    def k(rows_hbm, dst_hbm, out_hbm,
          dstseg, ids, offs, ost0, ost1, rb0, rb1, accum, sem0, sem1):
        cid = lax.axis_index("c")
        sid = lax.axis_index("s")
        ebase = sid * ept
        ostages = (ost0, ost1)
        rowbufs = (rb0, rb1)
        sems = (sem0, sem1)

        for (lo_c, size), owner in zip(chunks, owners):

            @pl.when(cid == owner)
            def _(lo_c=lo_c, size=size):
                def zb_step(i, c):
                    rb0[i // 8, pl.ds((i % 8) * 16, 16)] = jnp.zeros((16,), F32)
                    return c

                lax.fori_loop(0, _GC * 8, zb_step, 0)
                z = size // _NS
                zoff = sid * z
                o = 0
                while o < z:
                    ln = min(_GC, z - o)
                    pltpu.sync_copy(rb0.at[pl.ds(0, ln)],
                                    accum.at[pl.ds(zoff + o, ln)])
                    o += ln
                plsc.subcore_barrier()

                trash = cap - 16
                lane = lax.broadcasted_iota(I32, (16,), 0)
                dummy = jnp.full((16,), size, I32)

                def gat(t, b):
                    return pltpu.make_async_copy(
                        rows_hbm.at[ids.at[pl.ds(t * _GC, _GC)]],
                        rowbufs[b], sems[b])

                def flush(cnt):
                    for j in range(_GC // 16):
                        ppos = lane + (cnt + j * 16)
                        plsc.store_scatter(ids, [ppos], jnp.zeros((16,), I32))
                        plsc.store_scatter(offs, [ppos], dummy)

                    trips = (cnt + _GC - 1) // _GC

                    @pl.when(trips > 0)
                    def _():
                        gat(0, 0).start()

                    def gs_outer(g, c):
                        for b in range(2):
                            t = g * 2 + b

                            @pl.when(t < trips)
                            def _(t=t, b=b):
                                gat(t, b).wait()

                                @pl.when(t + 1 < trips)
                                def _():
                                    gat(t + 1, 1 - b).start()

                                for j in range(_GC // 16):
                                    ostages[b][pl.ds(j * 16, 16)] = offs[
                                        pl.ds(t * _GC + j * 16, 16)]
                                pltpu.sync_copy(rowbufs[b],
                                                accum.at[ostages[b]],
                                                add=True)
                        return c

                    lax.fori_loop(0, (trips + 1) // 2, gs_outer, 0)

                def seg_step(si, cnt):
                    pltpu.sync_copy(
                        dst_hbm.at[pl.ds(ebase + si * seg, seg)], dstseg)

                    def scan_step(v, cnt):
                        dv = dstseg[pl.ds(v * 16, 16)]
                        m = (dv >= lo_c) & (dv < lo_c + size)
                        mi = m.astype(I32)
                        incl = plsc.cumsum(mi)
                        pos = jnp.where(m, cnt + incl - mi, trash)
                        gid = lane + (ebase + si * seg + v * 16)
                        plsc.store_scatter(ids, [pos], gid)
                        plsc.store_scatter(offs, [pos], dv - lo_c)
                        return cnt + jnp.sum(mi)

                    cnt = lax.fori_loop(0, seg // 16, scan_step, cnt)
                    do_flush = cnt >= cap - seg - _GC

                    @pl.when(do_flush)
                    def _():
                        flush(cnt)

                    return jnp.where(do_flush, 0, cnt)

                cnt = lax.fori_loop(0, ept // seg, seg_step, 0)

                @pl.when(cnt > 0)
                def _():
                    flush(cnt)

                plsc.subcore_barrier()

                w = size // _NS
                pltpu.sync_copy(accum.at[pl.ds(sid * w, w)],
                                out_hbm.at[pl.ds(lo_c + sid * w, w)])
                plsc.subcore_barrier()

    if n_pad == n_rows:
        return k
    return lambda rows, dst: k(rows, dst)[:n_rows]


def _mgn(x, e, src, dst, p, i):
    w1 = p["eW1"][i]
    ws, wd, we = w1[:D], w1[D:2 * D], w1[2 * D:]
    pt, qt = _tc_pq(x, ws, wd)
    gp, gq = _sc_gather2(e.shape[0])(pt, qt, src, dst)
    e_new = _tc_edge(gp, gq, e, we, p["eW2"][i],
                     p["eb1"][i].reshape(1, D), p["eb2"][i].reshape(1, D),
                     p["eg"][i].reshape(1, D), p["ebt"][i].reshape(1, D))
    agg = _sc_scatter_add(e.shape[0], x.shape[0])(e_new, dst)
    nw1 = p["nW1"][i]
    x_new = _tc_node(x, agg, nw1[:D], nw1[D:], p["nW2"][i],
                     p["nb1"][i].reshape(1, D), p["nb2"][i].reshape(1, D),
                     p["ng"][i].reshape(1, D), p["nbt"][i].reshape(1, D))
    return x_new, e_new


def kernel(h_atm, h_bnd, h_ang, params_A, params_G, edge_index_G, edge_index_A):
    srcA, dstA = edge_index_A[0], edge_index_A[1]
    srcG, dstG = edge_index_G[0], edge_index_G[1]
    for i in range(2):
        h_bnd, h_ang = _mgn(h_bnd, h_ang, srcA, dstA, params_A, i)
        h_atm, h_bnd = _mgn(h_atm, h_bnd, srcG, dstG, params_G, i)
    return (h_atm, h_bnd, h_ang)
